# Initial kernel scaffold; baseline (speedup 1.0000x reference)
import jax
import jax.numpy as jnp
from jax.experimental import pallas as pl


def kernel(inputs):
    raise NotImplementedError("write your pallas kernel here")


# speedup vs baseline: 30.9965x; 30.9965x over previous
import functools

import jax
import jax.numpy as jnp
from jax import lax
from jax.experimental import pallas as pl
from jax.experimental.pallas import tpu as pltpu
from jax.experimental.pallas import tpu_sc as plsc

_NBINS = 4096
_NC, _NS, _L = 2, 16, 16
_NW = _NC * _NS
_MAX_PERCENTILE = 0.999999
_MIN_PERCENTILE = 0.999999


def _mm_body(x_ref, mn_ref, mx_ref):
    i = pl.program_id(0)
    b = x_ref[...]
    bmn = jnp.min(b)
    bmx = jnp.max(b)

    @pl.when(i == 0)
    def _():
        mn_ref[0, 0] = bmn
        mx_ref[0, 0] = bmx

    @pl.when(i != 0)
    def _():
        mn_ref[0, 0] = jnp.minimum(mn_ref[0, 0], bmn)
        mx_ref[0, 0] = jnp.maximum(mx_ref[0, 0], bmx)


def _minmax_tc(flat):
    rows = 4096
    cols = flat.shape[0] // rows
    x2 = flat.reshape(rows, cols)
    grid = 16
    blk = rows // grid
    mn, mx = pl.pallas_call(
        _mm_body,
        grid=(grid,),
        in_specs=[pl.BlockSpec((blk, cols), lambda i: (i, 0))],
        out_specs=[
            pl.BlockSpec(memory_space=pltpu.SMEM),
            pl.BlockSpec(memory_space=pltpu.SMEM),
        ],
        out_shape=[
            jax.ShapeDtypeStruct((1, 1), jnp.float32),
            jax.ShapeDtypeStruct((1, 1), jnp.float32),
        ],
    )(x2)
    return mn[0, 0], mx[0, 0]


def _hist_sc(flat, mn_vec, w_vec):
    n = flat.shape[0]
    pw = n // _NW
    ch = 32768
    nbuf = 2
    n_chunks = pw // ch
    assert pw % ch == 0

    mesh = plsc.VectorSubcoreMesh(
        core_axis_name="c", subcore_axis_name="s",
        num_cores=_NC, num_subcores=_NS)

    @functools.partial(
        pl.kernel,
        out_type=jax.ShapeDtypeStruct((_NW, _NBINS), jnp.float32),
        mesh=mesh,
        compiler_params=pltpu.CompilerParams(
            use_tc_tiling_on_sc=False, needs_layout_passes=False),
        scratch_types=[
            pltpu.VMEM((nbuf * ch,), jnp.float32),
            pltpu.VMEM((_NBINS,), jnp.float32),
            pltpu.VMEM((_L,), jnp.float32),
            pltpu.VMEM((_L,), jnp.float32),
            pltpu.SemaphoreType.DMA,
            pltpu.SemaphoreType.DMA,
        ],
    )
    def k(data_hbm, mn_hbm, w_hbm, out_hbm, buf, hist, mnv, wv, sem0, sem1):
        cid = lax.axis_index("c")
        sid = lax.axis_index("s")
        wid = sid * _NC + cid
        base = wid * pw

        pltpu.sync_copy(mn_hbm, mnv)
        pltpu.sync_copy(w_hbm, wv)

        zero = jnp.zeros((_L,), jnp.float32)

        def zbody(i, _):
            hist[pl.ds(i * _L, _L)] = zero
            return 0

        lax.fori_loop(0, _NBINS // _L, zbody, 0)

        sems = (sem0, sem1)

        def chunk_copy(c, b):
            return pltpu.make_async_copy(
                data_hbm.at[pl.ds(base + c * ch, ch)],
                buf.at[pl.ds(b * ch, ch)],
                sems[b])

        for b in range(nbuf):
            chunk_copy(b, b).start()

        mnb = mnv[...]
        wb = wv[...]
        ones = jnp.full((_L,), 1.0, jnp.float32)
        nb4096 = jnp.full((_L,), float(_NBINS), jnp.float32)
        izero = jnp.zeros((_L,), jnp.int32)
        imax = jnp.full((_L,), _NBINS - 1, jnp.int32)

        def inner(i, b0):
            v = buf[pl.ds(b0 + i * _L, _L)]
            t = (v - mnb) / wb * nb4096
            ii = jnp.minimum(jnp.maximum(t.astype(jnp.int32), izero), imax)
            plsc.addupdate_scatter(hist, [ii], ones)
            return b0

        def outer(c0, _):
            for b in range(nbuf):
                c = c0 * nbuf + b
                chunk_copy(c, b).wait()
                lax.fori_loop(0, ch // _L, inner, b * ch)

                @pl.when(c + nbuf < n_chunks)
                def _():
                    chunk_copy(c + nbuf, b).start()
            return 0

        lax.fori_loop(0, n_chunks // nbuf, outer, 0)

        pltpu.sync_copy(hist, out_hbm.at[wid])

    return k(flat, mn_vec, w_vec)


def kernel(inputs):
    flat = inputs.reshape(-1)
    mn, mx = _minmax_tc(flat)
    width = jnp.maximum(mx - mn, jnp.float32(1e-12))
    mn_vec = jnp.full((_L,), mn, jnp.float32)
    w_vec = jnp.full((_L,), width, jnp.float32)

    partials = _hist_sc(flat, mn_vec, w_vec)
    hist = partials.sum(axis=0)

    n = flat.shape[0]
    cdf = jnp.cumsum(hist)
    hi_bin = jnp.searchsorted(cdf, jnp.float32(_MAX_PERCENTILE) * n)
    lo_bin = jnp.searchsorted(cdf, jnp.float32(1.0 - _MIN_PERCENTILE) * n)
    edges = mn + width * jnp.arange(_NBINS + 1, dtype=jnp.float32) / _NBINS
    clip_max = edges[jnp.minimum(hi_bin + 1, _NBINS)]
    clip_min = edges[jnp.minimum(lo_bin, _NBINS)]
    scale = jnp.maximum((clip_max - clip_min) / 255.0, jnp.float32(1e-12))
    offset = jnp.round(-clip_min / scale) - 128.0
    return (inputs, scale, offset)
